# Initial kernel scaffold; baseline (speedup 1.0000x reference)
import jax
import jax.numpy as jnp
from jax.experimental import pallas as pl


def kernel(visual_features, zone_idx, month_enc, zone_table, Wm, bm, gamma, beta):
    raise NotImplementedError("write your pallas kernel here")


# speedup vs baseline: 1.7776x; 1.7776x over previous
import functools

import jax
import jax.numpy as jnp
from jax import lax
from jax.experimental import pallas as pl
from jax.experimental.pallas import tpu as pltpu
from jax.experimental.pallas import tpu_sc as plsc

_B = 16384
_VD = 2048
_ZE = 64
_MP = 64
_D = _VD + _ZE + _MP
_TB = 256


_GW = 128


def _zone_gather_sc(zone_table_padded, zone_idx):
    info = plsc.get_sparse_core_info()
    num_workers = info.num_cores * info.num_subcores
    b_per_w = _B // num_workers
    mesh = plsc.VectorSubcoreMesh(core_axis_name="c", subcore_axis_name="s")

    @functools.partial(
        pl.kernel,
        mesh=mesh,
        out_type=jax.ShapeDtypeStruct((_B, _GW), jnp.float32),
        scratch_types=[
            pltpu.VMEM((b_per_w,), jnp.int32),
            pltpu.VMEM((b_per_w, _GW), jnp.float32),
            pltpu.SemaphoreType.DMA,
        ],
    )
    def gather(table_hbm, idx_hbm, out_hbm, idx_v, rows_v, sem):
        wid = lax.axis_index("s") * info.num_cores + lax.axis_index("c")
        base = wid * b_per_w
        pltpu.sync_copy(idx_hbm.at[pl.ds(base, b_per_w)], idx_v)
        pltpu.async_copy(table_hbm.at[idx_v], rows_v, sem).wait()
        pltpu.sync_copy(rows_v, out_hbm.at[pl.ds(base, b_per_w)])

    return gather(zone_table_padded, zone_idx)


def _fused_body(v_ref, ze_ref, me_ref, wm_ref, bm_ref, g_ref, b_ref, o_ref):
    v = v_ref[...]
    ze = ze_ref[:, :_ZE]
    me = me_ref[...]
    wm = wm_ref[...]
    m = me[:, 0:1] * wm[0:1, :] + me[:, 1:2] * wm[1:2, :] + bm_ref[...]
    m = jnp.maximum(m, 0.0)
    tail = jnp.concatenate([ze, m], axis=1)
    inv_d = 1.0 / _D
    mean = (jnp.sum(v, axis=1, keepdims=True)
            + jnp.sum(tail, axis=1, keepdims=True)) * inv_d
    dv = v - mean
    dt = tail - mean
    var = (jnp.sum(dv * dv, axis=1, keepdims=True)
           + jnp.sum(dt * dt, axis=1, keepdims=True)) * inv_d
    r = lax.rsqrt(var + 1e-5)
    g = g_ref[...]
    b = b_ref[...]
    o_ref[:, :_VD] = dv * r * g[:, :_VD] + b[:, :_VD]
    o_ref[:, _VD:] = dt * r * g[:, _VD:] + b[:, _VD:]


def _fused_call(v, ze, me, wm, bm2, g2, b2):
    return pl.pallas_call(
        _fused_body,
        grid=(_B // _TB,),
        in_specs=[
            pl.BlockSpec((_TB, _VD), lambda i: (i, 0)),
            pl.BlockSpec((_TB, _GW), lambda i: (i, 0)),
            pl.BlockSpec((_TB, 2), lambda i: (i, 0)),
            pl.BlockSpec((2, _MP), lambda i: (0, 0)),
            pl.BlockSpec((1, _MP), lambda i: (0, 0)),
            pl.BlockSpec((1, _D), lambda i: (0, 0)),
            pl.BlockSpec((1, _D), lambda i: (0, 0)),
        ],
        out_specs=pl.BlockSpec((_TB, _D), lambda i: (i, 0)),
        out_shape=jax.ShapeDtypeStruct((_B, _D), jnp.float32),
        compiler_params=pltpu.CompilerParams(
            dimension_semantics=("arbitrary",),
        ),
    )(v, ze, me, wm, bm2, g2, b2)


def kernel(visual_features, zone_idx, month_enc, zone_table, Wm, bm, gamma, beta):
    table_padded = jnp.pad(zone_table, ((0, 0), (0, _GW - _ZE)))
    ze = _zone_gather_sc(table_padded, zone_idx)
    return _fused_call(
        visual_features,
        ze,
        month_enc,
        Wm,
        bm.reshape(1, _MP),
        gamma.reshape(1, _D),
        beta.reshape(1, _D),
    )
